# baseline (speedup 1.0000x reference)
import jax
import jax.numpy as jnp
from jax import lax
from jax.experimental import pallas as pl
from jax.experimental.pallas import tpu as pltpu
from jax.experimental.pallas import tpu_sc as plsc

_B, _H, _MAXS, _D = 8, 16, 2048, 128
_Q = 16
_NBH = _B * _H
_NC, _NS = 2, 16
_NW = _NC * _NS
_PAIRS_PER_W = _NBH // _NW


def _sc_scatter(pos_hbm, kval_hbm, vval_hbm, kout_ref, vout_ref,
                pos_v, src_v, sem):
    wid = lax.axis_index("s") * _NC + lax.axis_index("c")
    pltpu.sync_copy(pos_hbm, pos_v)
    pos = pos_v[...]
    for val_hbm, out_ref in ((kval_hbm, kout_ref), (vval_hbm, vout_ref)):
        for j in range(_PAIRS_PER_W):
            pair = wid * _PAIRS_PER_W + j
            pltpu.sync_copy(val_hbm.at[pl.ds(pair * _Q, _Q), :], src_v)
            idx = pos + pair * _MAXS
            pltpu.async_copy(src_v, out_ref.at[idx], sem).wait()


def kernel(k_cache, v_cache, input_pos, k_val, v_val):
    k_ref = jax.new_ref(k_cache.reshape(_NBH * _MAXS, _D))
    v_ref = jax.new_ref(v_cache.reshape(_NBH * _MAXS, _D))
    run = pl.kernel(
        _sc_scatter,
        out_type=(),
        mesh=plsc.VectorSubcoreMesh(core_axis_name="c", subcore_axis_name="s"),
        scratch_types=[
            pltpu.VMEM((_Q,), jnp.int32),
            pltpu.VMEM((_Q, _D), jnp.float32),
            pltpu.SemaphoreType.DMA,
        ],
    )
    run(input_pos, k_val.reshape(_NBH * _Q, _D), v_val.reshape(_NBH * _Q, _D),
        k_ref, v_ref)
    k_out = jax.freeze(k_ref).reshape(_B, _H, _MAXS, _D)
    v_out = jax.freeze(v_ref).reshape(_B, _H, _MAXS, _D)
    return (k_out, v_out)
